# baseline (speedup 1.0000x reference)
import functools

import jax
import jax.numpy as jnp
from jax import lax
from jax.experimental import pallas as pl
from jax.experimental.pallas import tpu as pltpu
from jax.experimental.pallas import tpu_sc as plsc

N = 10000
E = 320000
D = 128

NC = 2
NS = 16
L = 16

CHUNK = 64
EP = 327680
NSTAGE = 4
EDGES_PER_TILE = EP // NS
CH = EDGES_PER_TILE // NSTAGE // CHUNK
STG = CH * CHUNK // 2
ACC_ROWS = 10240
ROWS_PER_TILE = ACC_ROWS // NS
ZCH = ROWS_PER_TILE // CHUNK


@functools.partial(
    pl.kernel,
    out_type=jax.ShapeDtypeStruct((ACC_ROWS, D), jnp.float32),
    mesh=plsc.VectorSubcoreMesh(core_axis_name="c", subcore_axis_name="s"),
    scratch_types=[
        pltpu.VMEM((STG,), jnp.int32),
        pltpu.VMEM((STG,), jnp.int32),
        pltpu.VMEM((STG,), jnp.int32),
        pltpu.VMEM((2, CHUNK), jnp.int32),
        pltpu.VMEM((2, CHUNK), jnp.int32),
        pltpu.VMEM((2, CHUNK, D), jnp.float32),
        pltpu.VMEM((2, CHUNK, D), jnp.float32),
        pltpu.VMEM_SHARED((ACC_ROWS, D), jnp.float32),
        pltpu.SemaphoreType.DMA,
        pltpu.SemaphoreType.DMA,
        pltpu.SemaphoreType.DMA,
        pltpu.SemaphoreType.DMA,
        pltpu.SemaphoreType.DMA,
    ],
    compiler_params=pltpu.CompilerParams(needs_layout_passes=False),
)
def _spmm_sc(x_hbm, sdw_hbm, out_hbm,
             srcp_v, dstp_v, wp_v, gi_v, si_v, rows_v, sbuf_v, acc_sh,
             gsem0, gsem1, ssem0, ssem1, bsem):
    c = lax.axis_index("c")
    s = lax.axis_index("s")
    gsem = (gsem0, gsem1)
    ssem = (ssem0, ssem1)

    def build_idx(packed_ref, out_ring, b, g):
        base = g * (CHUNK // 2)
        for k in range(CHUNK // 32):
            v = packed_ref[pl.ds(base + k * 16, 16)]
            out_ring[b, pl.ds(k * 32, 16)] = v & 0xFFFF
            out_ring[b, pl.ds(k * 32 + 16, 16)] = v >> 16

    def start_gather(g, b):
        build_idx(srcp_v, gi_v, b, g)
        pltpu.async_copy(x_hbm.at[gi_v.at[b]], rows_v.at[b], gsem[b])

    def wait_gather(b):
        pltpu.make_async_copy(
            x_hbm.at[gi_v.at[b]], rows_v.at[b], gsem[b]).wait()

    @pl.when(c == 0)
    def _():
        def zrow(i, _):
            for j in range(D // L):
                sbuf_v[0, i, pl.ds(j * L, L)] = jnp.zeros((L,), jnp.float32)
            return 0
        lax.fori_loop(0, CHUNK, zrow, 0)

        row0 = s * ROWS_PER_TILE
        for k in range(ZCH):
            pltpu.async_copy(
                sbuf_v.at[0], acc_sh.at[pl.ds(row0 + k * CHUNK, CHUNK)],
                ssem0)
        for k in range(ZCH):
            pltpu.make_async_copy(
                sbuf_v.at[0], acc_sh.at[pl.ds(row0 + k * CHUNK, CHUNK)],
                ssem0).wait()
        plsc.subcore_barrier()

        for stage in range(NSTAGE):
            pltpu.async_copy(sdw_hbm.at[0, s, stage], srcp_v, gsem0)
            pltpu.async_copy(sdw_hbm.at[1, s, stage], dstp_v, gsem1)
            pltpu.async_copy(sdw_hbm.at[2, s, stage], wp_v, bsem)
            pltpu.make_async_copy(
                sdw_hbm.at[0, s, stage], srcp_v, gsem0).wait()
            pltpu.make_async_copy(
                sdw_hbm.at[1, s, stage], dstp_v, gsem1).wait()
            pltpu.make_async_copy(sdw_hbm.at[2, s, stage], wp_v, bsem).wait()

            start_gather(0, 0)
            start_gather(1, 1)

            def pair_body(p, _):
                for b in range(2):
                    g = 2 * p + b
                    wait_gather(b)

                    @pl.when(g > 1)
                    def _():
                        pltpu.make_async_copy(
                            sbuf_v.at[b], acc_sh.at[si_v.at[b]],
                            ssem[b]).wait()

                    build_idx(dstp_v, si_v, b, g)

                    @plsc.parallel_loop(0, CHUNK // 2, step=1, unroll=2)
                    def _(t):
                        wv = plsc.load_gather(
                            wp_v, [jnp.full((L,), g * (CHUNK // 2) + t,
                                            jnp.int32)])
                        wlo = plsc.bitcast(wv << 16, jnp.float32)
                        whi = plsc.bitcast(
                            wv & jnp.int32(-65536), jnp.float32)
                        rlo = ((t >> 4) << 5) + (t & 15)
                        for j in range(D // L):
                            sbuf_v[b, rlo, pl.ds(j * L, L)] = (
                                rows_v[b, rlo, pl.ds(j * L, L)] * wlo)
                        for j in range(D // L):
                            sbuf_v[b, rlo + L, pl.ds(j * L, L)] = (
                                rows_v[b, rlo + L, pl.ds(j * L, L)] * whi)

                    pltpu.async_copy(
                        sbuf_v.at[b], acc_sh.at[si_v.at[b]], ssem[b],
                        add=True)

                    @pl.when(g + 2 < CH)
                    def _():
                        start_gather(g + 2, b)
                return 0
            lax.fori_loop(0, CH // 2, pair_body, 0)

            for b in range(2):
                pltpu.make_async_copy(
                    sbuf_v.at[b], acc_sh.at[si_v.at[b]], ssem[b]).wait()

        plsc.subcore_barrier()

        for k in range(ZCH):
            r = row0 + k * CHUNK
            pltpu.async_copy(acc_sh.at[pl.ds(r, CHUNK)],
                             out_hbm.at[pl.ds(r, CHUNK)], ssem0)
        for k in range(ZCH):
            r = row0 + k * CHUNK
            pltpu.make_async_copy(acc_sh.at[pl.ds(r, CHUNK)],
                                  out_hbm.at[pl.ds(r, CHUNK)], ssem0).wait()


def _dense_body(x_ref, acc_ref, w0_ref, w1_ref, b0_ref, b1_ref,
                s0_ref, s1_ref, o0_ref, o1_ref, out_ref):
    def trans(f, w_ref, b, sc, off):
        h = lax.dot_general(f, w_ref[...], (((1,), (1,)), ((), ())),
                            preferred_element_type=jnp.float32,
                            precision=lax.Precision.HIGHEST)
        h = jnp.maximum(h + b, 0.0)
        mean = jnp.mean(h, axis=1, keepdims=True)
        cent = h - mean
        var = jnp.mean(cent * cent, axis=1, keepdims=True) + 1e-9
        return cent * sc * jax.lax.rsqrt(var) + off

    out_ref[...] = (trans(x_ref[...], w0_ref, b0_ref[...], s0_ref[...], o0_ref[...])
                    + trans(acc_ref[...], w1_ref, b1_ref[...], s1_ref[...], o1_ref[...]))


_BLK = 400


def _dense_tc(x, acc, W0, W1, b0, b1, scale0, scale1, offset0, offset1):
    grid = (N // _BLK,)
    vec_spec = pl.BlockSpec((1, D), lambda i: (0, 0))
    return pl.pallas_call(
        _dense_body,
        grid=grid,
        in_specs=[
            pl.BlockSpec((_BLK, D), lambda i: (i, 0)),
            pl.BlockSpec((_BLK, D), lambda i: (i, 0)),
            pl.BlockSpec((D, D), lambda i: (0, 0)),
            pl.BlockSpec((D, D), lambda i: (0, 0)),
            vec_spec, vec_spec, vec_spec, vec_spec, vec_spec, vec_spec,
        ],
        out_specs=pl.BlockSpec((_BLK, D), lambda i: (i, 0)),
        out_shape=jax.ShapeDtypeStruct((N, D), jnp.float32),
    )(x, acc, W0, W1, b0.reshape(1, D), b1.reshape(1, D),
      scale0.reshape(1, D), scale1.reshape(1, D),
      offset0.reshape(1, D), offset1.reshape(1, D))


def kernel(x, edge_index, edge_weight, W0, W1, b0, b1,
           scale0, scale1, offset0, offset1):
    dst = edge_index[0]
    src = edge_index[1]
    pad = EP - E

    src_f = jnp.concatenate([src, jnp.zeros((pad,), jnp.int32)])
    dst_f = jnp.concatenate(
        [dst, N + (jnp.arange(pad, dtype=jnp.int32) % 16)])
    w_i = lax.bitcast_convert_type(
        jnp.concatenate([edge_weight, jnp.zeros((pad,), jnp.float32)]
                        ).astype(jnp.bfloat16), jnp.uint16).astype(jnp.int32)

    sdw = jnp.stack([src_f, dst_f, w_i]).reshape(3, NS, NSTAGE, -1, 2, 16)
    sdw = (sdw[..., 0, :] | (sdw[..., 1, :] << 16)).reshape(
        3, NS, NSTAGE, STG)

    acc = _spmm_sc(x, sdw)
    return _dense_tc(x, acc, W0, W1, b0, b1,
                     scale0, scale1, offset0, offset1)


# speedup vs baseline: 1.0970x; 1.0970x over previous
import functools

import jax
import jax.numpy as jnp
from jax import lax
from jax.experimental import pallas as pl
from jax.experimental.pallas import tpu as pltpu
from jax.experimental.pallas import tpu_sc as plsc

N = 10000
E = 320000
D = 128

NC = 2
NS = 16
L = 16

CHUNK = 64
EP = 327680
CH0 = 160
CH1 = 160
E0 = NS * CH0 * CHUNK
STG = CH0 * CHUNK // 2
ACC_ROWS = 10240
ROWS_PER_TILE = ACC_ROWS // NS
ZCH = ROWS_PER_TILE // CHUNK


@functools.partial(
    pl.kernel,
    out_type=jax.ShapeDtypeStruct((NC, ACC_ROWS, D), jnp.float32),
    mesh=plsc.VectorSubcoreMesh(core_axis_name="c", subcore_axis_name="s"),
    scratch_types=[
        pltpu.VMEM((STG,), jnp.int32),
        pltpu.VMEM((STG,), jnp.int32),
        pltpu.VMEM((STG,), jnp.int32),
        pltpu.VMEM((2, CHUNK), jnp.int32),
        pltpu.VMEM((2, CHUNK), jnp.int32),
        pltpu.VMEM((2, CHUNK, D), jnp.float32),
        pltpu.VMEM((2, CHUNK, D), jnp.float32),
        pltpu.VMEM_SHARED((ACC_ROWS, D), jnp.float32),
        pltpu.SemaphoreType.DMA,
        pltpu.SemaphoreType.DMA,
        pltpu.SemaphoreType.DMA,
        pltpu.SemaphoreType.DMA,
        pltpu.SemaphoreType.DMA,
    ],
    compiler_params=pltpu.CompilerParams(needs_layout_passes=False),
)
def _spmm_sc(x_hbm, sdw_hbm, out_hbm,
             srcp_v, dstp_v, wp_v, gi_v, si_v, rows_v, sbuf_v, acc_sh,
             gsem0, gsem1, ssem0, ssem1, bsem):
    c = lax.axis_index("c")
    s = lax.axis_index("s")
    gsem = (gsem0, gsem1)
    ssem = (ssem0, ssem1)
    nch = jnp.where(c == 0, CH0, CH1)

    def zrow(i, _):
        for j in range(D // L):
            sbuf_v[0, i, pl.ds(j * L, L)] = jnp.zeros((L,), jnp.float32)
        return 0
    lax.fori_loop(0, CHUNK, zrow, 0)

    row0 = s * ROWS_PER_TILE
    pltpu.async_copy(sdw_hbm.at[0, c, s], srcp_v, gsem0)
    pltpu.async_copy(sdw_hbm.at[1, c, s], dstp_v, gsem1)
    pltpu.async_copy(sdw_hbm.at[2, c, s], wp_v, bsem)
    for k in range(ZCH):
        pltpu.async_copy(
            sbuf_v.at[0], acc_sh.at[pl.ds(row0 + k * CHUNK, CHUNK)], ssem0)
    pltpu.make_async_copy(sdw_hbm.at[0, c, s], srcp_v, gsem0).wait()
    pltpu.make_async_copy(sdw_hbm.at[1, c, s], dstp_v, gsem1).wait()
    pltpu.make_async_copy(sdw_hbm.at[2, c, s], wp_v, bsem).wait()
    for k in range(ZCH):
        pltpu.make_async_copy(
            sbuf_v.at[0], acc_sh.at[pl.ds(row0 + k * CHUNK, CHUNK)],
            ssem0).wait()
    plsc.subcore_barrier()

    def build_idx(packed_ref, out_ring, b, g):
        base = g * (CHUNK // 2)
        for k in range(CHUNK // 32):
            v = packed_ref[pl.ds(base + k * 16, 16)]
            out_ring[b, pl.ds(k * 32, 16)] = v & 0xFFFF
            out_ring[b, pl.ds(k * 32 + 16, 16)] = v >> 16

    def start_gather(g, b):
        build_idx(srcp_v, gi_v, b, g)
        pltpu.async_copy(x_hbm.at[gi_v.at[b]], rows_v.at[b], gsem[b])

    def wait_gather(b):
        pltpu.make_async_copy(
            x_hbm.at[gi_v.at[b]], rows_v.at[b], gsem[b]).wait()

    start_gather(0, 0)
    start_gather(1, 1)

    def pair_body(p, _):
        for b in range(2):
            g = 2 * p + b
            wait_gather(b)

            @pl.when(g > 1)
            def _():
                pltpu.make_async_copy(
                    sbuf_v.at[b], acc_sh.at[si_v.at[b]], ssem[b]).wait()

            build_idx(dstp_v, si_v, b, g)

            @plsc.parallel_loop(0, CHUNK // 2, step=1, unroll=2)
            def _(t):
                wv = plsc.load_gather(
                    wp_v, [jnp.full((L,), g * (CHUNK // 2) + t, jnp.int32)])
                wlo = plsc.bitcast(wv << 16, jnp.float32)
                whi = plsc.bitcast(wv & jnp.int32(-65536), jnp.float32)
                rlo = ((t >> 4) << 5) + (t & 15)
                for j in range(D // L):
                    sbuf_v[b, rlo, pl.ds(j * L, L)] = (
                        rows_v[b, rlo, pl.ds(j * L, L)] * wlo)
                for j in range(D // L):
                    sbuf_v[b, rlo + L, pl.ds(j * L, L)] = (
                        rows_v[b, rlo + L, pl.ds(j * L, L)] * whi)

            pltpu.async_copy(
                sbuf_v.at[b], acc_sh.at[si_v.at[b]], ssem[b], add=True)

            @pl.when(g + 2 < nch)
            def _():
                start_gather(g + 2, b)
        return 0
    lax.fori_loop(0, nch // 2, pair_body, 0)

    for b in range(2):
        pltpu.make_async_copy(
            sbuf_v.at[b], acc_sh.at[si_v.at[b]], ssem[b]).wait()
    plsc.subcore_barrier()

    for k in range(ZCH):
        r = row0 + k * CHUNK
        pltpu.async_copy(acc_sh.at[pl.ds(r, CHUNK)],
                         out_hbm.at[c, pl.ds(r, CHUNK)], ssem0)
    for k in range(ZCH):
        r = row0 + k * CHUNK
        pltpu.make_async_copy(acc_sh.at[pl.ds(r, CHUNK)],
                              out_hbm.at[c, pl.ds(r, CHUNK)], ssem0).wait()


def _dense_body(x_ref, acc_ref, w0_ref, w1_ref, b0_ref, b1_ref,
                s0_ref, s1_ref, o0_ref, o1_ref, out_ref):
    h1 = acc_ref[0] + acc_ref[1]

    def trans(f, w_ref, b, sc, off):
        h = lax.dot_general(f, w_ref[...], (((1,), (1,)), ((), ())),
                            preferred_element_type=jnp.float32,
                            precision=lax.Precision.HIGHEST)
        h = jnp.maximum(h + b, 0.0)
        mean = jnp.mean(h, axis=1, keepdims=True)
        cent = h - mean
        var = jnp.mean(cent * cent, axis=1, keepdims=True) + 1e-9
        return cent * sc * jax.lax.rsqrt(var) + off

    out_ref[...] = (trans(x_ref[...], w0_ref, b0_ref[...], s0_ref[...], o0_ref[...])
                    + trans(h1, w1_ref, b1_ref[...], s1_ref[...], o1_ref[...]))


_BLK = 400


def _dense_tc(x, acc, W0, W1, b0, b1, scale0, scale1, offset0, offset1):
    grid = (N // _BLK,)
    vec_spec = pl.BlockSpec((1, D), lambda i: (0, 0))
    return pl.pallas_call(
        _dense_body,
        grid=grid,
        in_specs=[
            pl.BlockSpec((_BLK, D), lambda i: (i, 0)),
            pl.BlockSpec((NC, _BLK, D), lambda i: (0, i, 0)),
            pl.BlockSpec((D, D), lambda i: (0, 0)),
            pl.BlockSpec((D, D), lambda i: (0, 0)),
            vec_spec, vec_spec, vec_spec, vec_spec, vec_spec, vec_spec,
        ],
        out_specs=pl.BlockSpec((_BLK, D), lambda i: (i, 0)),
        out_shape=jax.ShapeDtypeStruct((N, D), jnp.float32),
    )(x, acc, W0, W1, b0.reshape(1, D), b1.reshape(1, D),
      scale0.reshape(1, D), scale1.reshape(1, D),
      offset0.reshape(1, D), offset1.reshape(1, D))


def kernel(x, edge_index, edge_weight, W0, W1, b0, b1,
           scale0, scale1, offset0, offset1):
    dst = edge_index[0]
    src = edge_index[1]
    pad = EP - E

    src_f = jnp.concatenate([src, jnp.zeros((pad,), jnp.int32)])
    dst_f = jnp.concatenate(
        [dst, N + (jnp.arange(pad, dtype=jnp.int32) % 16)])
    w_i = lax.bitcast_convert_type(
        jnp.concatenate([edge_weight, jnp.zeros((pad,), jnp.float32)]
                        ).astype(jnp.bfloat16), jnp.uint16).astype(jnp.int32)

    sdw = jnp.stack([src_f, dst_f, w_i])
    a0 = sdw[:, :E0].reshape(3, 1, NS, CH0 * CHUNK)
    a1 = sdw[:, E0:].reshape(3, 1, NS, (320 - CH0) * CHUNK)
    if CH1 < CH0:
        a1 = jnp.concatenate(
            [a1, jnp.zeros((3, 1, NS, (CH0 - CH1) * CHUNK), jnp.int32)],
            axis=-1)
    sdw = jnp.concatenate([a0, a1], axis=1).reshape(3, NC, NS, -1, 2, 16)
    sdw = (sdw[..., 0, :] | (sdw[..., 1, :] << 16)).reshape(3, NC, NS, STG)

    acc = _spmm_sc(x, sdw)
    return _dense_tc(x, acc, W0, W1, b0, b1,
                     scale0, scale1, offset0, offset1)
